# baseline (speedup 1.0000x reference)
import functools

import jax
import jax.numpy as jnp
from jax import lax
from jax.experimental import pallas as pl
from jax.experimental.pallas import tpu as pltpu
from jax.experimental.pallas import tpu_sc as plsc

_NQ, _NS, _E, _H = 256, 128, 256, 256
_BQ = 32
_NCHUNK = _H // 16
_ROWS_PER_W = 4


def _tc1(qf, sf, y_row, y_col, nW1, nb1, nW2, nb2, w1a, w1b, b1, ra, rb1,
         s_out, a_out, b_out, same_out, qa_out):
    s = jnp.dot(jnp.maximum(jnp.dot(sf[...], nW1[...],
                                    preferred_element_type=jnp.float32)
                            + nb1[...], 0.0), nW2[...],
                preferred_element_type=jnp.float32) + nb2[...]
    s_out[...] = s
    a_out[...] = jnp.dot(s, w1a[...], preferred_element_type=jnp.float32) + b1[...]
    b_out[...] = jnp.dot(s, w1b[...], preferred_element_type=jnp.float32)
    same_out[...] = (y_col[...] == y_row[...]).astype(jnp.float32)
    q = jnp.dot(jnp.maximum(jnp.dot(qf[...], nW1[...],
                                    preferred_element_type=jnp.float32)
                            + nb1[...], 0.0), nW2[...],
                preferred_element_type=jnp.float32) + nb2[...]
    qa_out[...] = jnp.dot(q, ra[...], preferred_element_type=jnp.float32) + rb1[...]


def _sc_round(a_hbm, b_hbm, s_hbm, same_hbm, r_hbm, nv_hbm,
              b_vm, s_vm, aown_vm, sown_vm, same_vm, r_vm, nv_vm):
    nc = plsc.get_sparse_core_info().num_cores
    wid = lax.axis_index("s") * nc + lax.axis_index("c")
    base = wid * _ROWS_PER_W

    pltpu.sync_copy(b_hbm, b_vm)
    pltpu.sync_copy(s_hbm, s_vm)
    pltpu.sync_copy(a_hbm.at[pl.ds(base, _ROWS_PER_W)], aown_vm)
    pltpu.sync_copy(s_hbm.at[pl.ds(base, _ROWS_PER_W)], sown_vm)
    pltpu.sync_copy(same_hbm.at[pl.ds(base, _ROWS_PER_W)], same_vm.at[:, 0:_NS])

    zero = jnp.zeros((16,), jnp.float32)

    def zbody(il, carry):
        def zc(c, cz):
            r_vm[il, pl.ds(pl.multiple_of(c * 16, 16), 16)] = zero
            return cz
        lax.fori_loop(0, _NCHUNK, zc, 0)
        nv_vm[il, :] = zero
        return carry

    lax.fori_loop(0, _ROWS_PER_W, zbody, 0)

    def ilbody(il, carry0):
        ig = base + il

        def jcbody(jc, carry1):
            j0 = pl.multiple_of(jc * 16, 16)
            m = same_vm[il, pl.ds(j0, 16)]
            cnt = m[0]
            for k in range(1, 16):
                cnt = cnt + m[k]

            @pl.when(cnt > 0.5)
            def _():
                for k in range(16):
                    j = j0 + k

                    @pl.when((m[k] > 0.5) & (j != ig))
                    def _(j=j):
                        def cmpbody(c, nm):
                            sl = pl.ds(pl.multiple_of(c * 16, 16), 16)
                            return nm + jnp.where(
                                sown_vm[il, sl] != s_vm[j, sl], 1.0, 0.0)
                        nm = lax.fori_loop(0, _NCHUNK, cmpbody,
                                           jnp.zeros((16,), jnp.float32))
                        ndiff = nm[0]
                        for c in range(1, 16):
                            ndiff = ndiff + nm[c]

                        @pl.when(ndiff > 0.5)
                        def _(j=j):
                            def accbody(c, ca):
                                sl = pl.ds(pl.multiple_of(c * 16, 16), 16)
                                t = jnp.maximum(
                                    aown_vm[il, sl] + b_vm[j, sl], 0.0)
                                r_vm[il, sl] = r_vm[il, sl] + t
                                return ca
                            lax.fori_loop(0, _NCHUNK, accbody, 0)
                            nv_vm[il, :] = nv_vm[il, :] + 1.0
            return carry1

        lax.fori_loop(0, _NS // 16, jcbody, 0)
        return carry0

    lax.fori_loop(0, _ROWS_PER_W, ilbody, 0)

    pltpu.sync_copy(r_vm, r_hbm.at[pl.ds(base, _ROWS_PER_W)])
    pltpu.sync_copy(nv_vm, nv_hbm.at[pl.ds(base, _ROWS_PER_W)])


_sc_round_call = functools.partial(
    pl.kernel,
    mesh=plsc.VectorSubcoreMesh(core_axis_name="c", subcore_axis_name="s"),
    out_type=[jax.ShapeDtypeStruct((_NS, _H), jnp.float32),
              jax.ShapeDtypeStruct((_NS, 16), jnp.float32)],
    scratch_types=[pltpu.VMEM((_NS, _H), jnp.float32),
                   pltpu.VMEM((_NS, _H), jnp.float32),
                   pltpu.VMEM((_ROWS_PER_W, _H), jnp.float32),
                   pltpu.VMEM((_ROWS_PER_W, _H), jnp.float32),
                   pltpu.VMEM((_ROWS_PER_W, _NS + 16), jnp.float32),
                   pltpu.VMEM((_ROWS_PER_W, _H), jnp.float32),
                   pltpu.VMEM((_ROWS_PER_W, 16), jnp.float32)],
)(_sc_round)


def _tc2(s_in, r_in, nv_in, same_in, W2, b2, w1a, w1b, b1,
         s_out, a_out, b_out):
    s = s_in[...]
    nv = nv_in[..., 0:1]
    cc = jnp.sum(same_in[...], axis=1, keepdims=True)
    agg = jnp.dot(r_in[...] / jnp.maximum(nv, 1.0), W2[...],
                  preferred_element_type=jnp.float32) + b2[...]
    s = jnp.where((cc > 1.0) & (nv > 0.0), s + agg, s)
    s_out[...] = s
    a_out[...] = jnp.dot(s, w1a[...], preferred_element_type=jnp.float32) + b1[...]
    b_out[...] = jnp.dot(s, w1b[...], preferred_element_type=jnp.float32)


def _tc3(s_in, r_in, nv_in, same_in, W2, b2, qa_in, rb, rw2, rb2, out):
    s = s_in[...]
    nv = nv_in[..., 0:1]
    cc = jnp.sum(same_in[...], axis=1, keepdims=True)
    agg = jnp.dot(r_in[...] / jnp.maximum(nv, 1.0), W2[...],
                  preferred_element_type=jnp.float32) + b2[...]
    s = jnp.where((cc > 1.0) & (nv > 0.0), s + agg, s)

    qA = qa_in[...]
    sB = jnp.dot(s, rb[...], preferred_element_type=jnp.float32)
    w2 = rw2[...]
    bias = rb2[0, 0]
    for r0 in range(0, _NQ, _BQ):
        T = jnp.maximum(qA[r0:r0 + _BQ][:, None, :] + sB[None, :, :], 0.0)
        out[r0:r0 + _BQ, :] = jnp.sum(T * w2[None, :, :], axis=-1) + bias


def _vmem_call(body, n_in, out_shapes):
    return pl.pallas_call(
        body,
        out_shape=[jax.ShapeDtypeStruct(s, jnp.float32) for s in out_shapes],
        in_specs=[pl.BlockSpec(memory_space=pltpu.VMEM) for _ in range(n_in)],
        out_specs=[pl.BlockSpec(memory_space=pltpu.VMEM) for _ in out_shapes],
    )


@jax.jit
def kernel(query_features, support_features, support_y,
           node_W1, node_b1, node_W2, node_b2,
           msg_W1, msg_b1, msg_W2, msg_b2,
           rel_W1, rel_b1, rel_W2, rel_b2):
    y_row = support_y.reshape(1, _NS)
    y_col = support_y.reshape(_NS, 1)

    s0, a0, b0, same, qa = _vmem_call(_tc1, 13,
        [(_NS, _H), (_NS, _H), (_NS, _H), (_NS, _NS), (_NQ, _H)])(
        query_features, support_features, y_row, y_col,
        node_W1, node_b1.reshape(1, _H), node_W2, node_b2.reshape(1, _H),
        msg_W1[0, :_H], msg_W1[0, _H:], msg_b1[0].reshape(1, _H),
        rel_W1[:_H], rel_b1.reshape(1, _H))

    r0, nv0 = _sc_round_call(a0, b0, s0, same)

    s1, a1, b1_ = _vmem_call(_tc2, 9, [(_NS, _H), (_NS, _H), (_NS, _H)])(
        s0, r0, nv0, same, msg_W2[0], msg_b2[0].reshape(1, _H),
        msg_W1[1, :_H], msg_W1[1, _H:], msg_b1[1].reshape(1, _H))

    r1, nv1 = _sc_round_call(a1, b1_, s1, same)

    return _vmem_call(_tc3, 10, [(_NQ, _NS)])(
        s1, r1, nv1, same, msg_W2[1], msg_b2[1].reshape(1, _H),
        qa, rel_W1[_H:], rel_W2.reshape(1, _H), rel_b2.reshape(1, 1))[0]


# speedup vs baseline: 4.4094x; 4.4094x over previous
import jax
import jax.numpy as jnp
from jax.experimental import pallas as pl
from jax.experimental.pallas import tpu as pltpu

_NQ, _NS, _E, _H = 256, 128, 256, 256
_BI = 32
_BQ = 32


def _dot(a, b):
    return jax.lax.dot_general(a, b, (((1,), (0,)), ((), ())),
                               preferred_element_type=jnp.float32)


def _body(qf, sf, y_row, y_col, nW1, nb1, nW2, nb2,
          m1a0, m1b0, mb10, mW20, mb20,
          m1a1, m1b1, mb11, mW21, mb21,
          ra, rb, rb1, rw2, rb2,
          out, s_ref, snew_ref):
    s = _dot(jnp.maximum(_dot(sf[...], nW1[...]) + nb1[...], 0.0), nW2[...]) + nb2[...]
    s_ref[...] = s
    q = _dot(jnp.maximum(_dot(qf[...], nW1[...]) + nb1[...], 0.0), nW2[...]) + nb2[...]

    same_f = (y_col[...] == y_row[...]).astype(jnp.float32)
    cc = jnp.sum(same_f, axis=1, keepdims=True)

    for (w1a, w1b, b1, W2, b2) in ((m1a0, m1b0, mb10, mW20, mb20),
                                   (m1a1, m1b1, mb11, mW21, mb21)):
        s = s_ref[...]
        si = jax.lax.bitcast_convert_type(s, jnp.int32)
        A = _dot(s, w1a[...]) + b1[...]
        B = _dot(s, w1b[...])
        for r0 in range(0, _NS, _BI):
            s_blk = s[r0:r0 + _BI]
            d = si[r0:r0 + _BI][:, None, :] ^ si[None, :, :]
            neq = (jnp.max(d, axis=-1) > 0) | (jnp.min(d, axis=-1) < 0)
            valid = same_f[r0:r0 + _BI] * neq.astype(jnp.float32)
            T = jnp.maximum(A[r0:r0 + _BI][:, None, :] + B[None, :, :], 0.0)
            R = jax.lax.dot_general(valid, T, (((1,), (1,)), ((0,), (0,))),
                                    preferred_element_type=jnp.float32)
            nv = jnp.sum(valid, axis=1, keepdims=True)
            agg = _dot(R / jnp.maximum(nv, 1.0), W2[...]) + b2[...]
            upd = (cc[r0:r0 + _BI] > 1.0) & (nv > 0.0)
            snew_ref[r0:r0 + _BI, :] = jnp.where(upd, s_blk + agg, s_blk)
        s_ref[...] = snew_ref[...]

    s = s_ref[...]
    qA = _dot(q, ra[...]) + rb1[...]
    sB = _dot(s, rb[...])
    w2 = rw2[...]
    bias = rb2[0, 0]
    for r0 in range(0, _NQ, _BQ):
        T = jnp.maximum(qA[r0:r0 + _BQ][:, None, :] + sB[None, :, :], 0.0)
        out[r0:r0 + _BQ, :] = jnp.sum(T * w2[None, :, :], axis=-1) + bias


@jax.jit
def kernel(query_features, support_features, support_y,
           node_W1, node_b1, node_W2, node_b2,
           msg_W1, msg_b1, msg_W2, msg_b2,
           rel_W1, rel_b1, rel_W2, rel_b2):
    y_row = support_y.reshape(1, _NS)
    y_col = support_y.reshape(_NS, 1)
    args = (
        query_features, support_features, y_row, y_col,
        node_W1, node_b1.reshape(1, _H), node_W2, node_b2.reshape(1, _H),
        msg_W1[0, :_H], msg_W1[0, _H:], msg_b1[0].reshape(1, _H),
        msg_W2[0], msg_b2[0].reshape(1, _H),
        msg_W1[1, :_H], msg_W1[1, _H:], msg_b1[1].reshape(1, _H),
        msg_W2[1], msg_b2[1].reshape(1, _H),
        rel_W1[:_H], rel_W1[_H:], rel_b1.reshape(1, _H),
        rel_W2.reshape(1, _H), rel_b2.reshape(1, 1),
    )
    return pl.pallas_call(
        _body,
        out_shape=jax.ShapeDtypeStruct((_NQ, _NS), jnp.float32),
        in_specs=[pl.BlockSpec(memory_space=pltpu.VMEM) for _ in args],
        out_specs=pl.BlockSpec(memory_space=pltpu.VMEM),
        scratch_shapes=[pltpu.VMEM((_NS, _H), jnp.float32),
                        pltpu.VMEM((_NS, _H), jnp.float32)],
    )(*args)
